# baseline (speedup 1.0000x reference)
import functools

import jax
import jax.numpy as jnp
from jax import lax
from jax.experimental import pallas as pl
from jax.experimental.pallas import tpu as pltpu
from jax.experimental.pallas import tpu_sc as plsc

_DIM = 1024
_SCALE = _DIM ** (-0.5)
_NC, _NS = 2, 16
_NW = _NC * _NS
_LANES = 16

_CHUNK_ROWS = 8
_COL_VREGS = _DIM // _LANES
_RING = 4


@functools.partial(jax.jit, static_argnums=0)
def _sc_scaled_copy(rows, emb):
    rows_per_w = rows // _NW
    n_chunks = rows_per_w // _CHUNK_ROWS
    assert n_chunks % _RING == 0 and n_chunks >= 2 * _RING

    mesh = plsc.VectorSubcoreMesh(
        core_axis_name="c", subcore_axis_name="s",
        num_cores=_NC, num_subcores=_NS)

    @functools.partial(
        pl.kernel,
        out_type=jax.ShapeDtypeStruct((rows, _DIM), jnp.float32),
        mesh=mesh,
        scratch_types=[
            pltpu.VMEM((_CHUNK_ROWS, _DIM), jnp.float32),
            pltpu.VMEM((_CHUNK_ROWS, _DIM), jnp.float32),
            pltpu.VMEM((_CHUNK_ROWS, _DIM), jnp.float32),
            pltpu.VMEM((_CHUNK_ROWS, _DIM), jnp.float32),
            pltpu.SemaphoreType.DMA,
            pltpu.SemaphoreType.DMA,
        ],
    )
    def scale_kernel(emb_hbm, out_hbm, b0, b1, b2, b3, sem_in, sem_out):
        wid = lax.axis_index("s") * _NC + lax.axis_index("c")
        base = wid * rows_per_w
        bufs = [b0, b1, b2, b3]

        def issue_in(c, b):
            pltpu.async_copy(
                emb_hbm.at[pl.ds(base + c * _CHUNK_ROWS, _CHUNK_ROWS)],
                bufs[b], sem_in)

        def issue_out(c, b):
            pltpu.async_copy(
                bufs[b], out_hbm.at[pl.ds(base + c * _CHUNK_ROWS, _CHUNK_ROWS)],
                sem_out)

        def wait_one(sem):
            pltpu.make_async_copy(
                emb_hbm.at[pl.ds(0, _CHUNK_ROWS)], bufs[0], sem).wait()

        def compute(b):
            buf = bufs[b]

            @plsc.parallel_loop(0, _COL_VREGS, unroll=16)
            def _(i):
                sl = pl.ds(i * _LANES, _LANES)
                for r in range(_CHUNK_ROWS):
                    buf[r, sl] = buf[r, sl] * _SCALE

        issue_in(0, 0)
        issue_in(1, 1)
        for c in range(_RING):
            if c >= 2:
                wait_one(sem_out)
            issue_in(c + 2, (c + 2) % _RING)
            wait_one(sem_in)
            compute(c % _RING)
            issue_out(c, c % _RING)

        def ring_body(g, carry):
            for b in range(_RING):
                c = _RING * g + b
                wait_one(sem_out)
                issue_in(c + 2, (b + 2) % _RING)
                wait_one(sem_in)
                compute(b)
                issue_out(c, b)
            return carry

        lax.fori_loop(1, n_chunks // _RING - 1, ring_body, 0)

        for c in range(n_chunks - _RING, n_chunks):
            b = c % _RING
            if c + 2 < n_chunks:
                wait_one(sem_out)
                issue_in(c + 2, (b + 2) % _RING)
            wait_one(sem_in)
            compute(b)
            issue_out(c, b)
        for _ in range(_RING):
            wait_one(sem_out)

    return scale_kernel(emb)


def kernel(x, emb):
    seq_len = x.shape[1]
    return _sc_scaled_copy(seq_len, emb[:seq_len])


# speedup vs baseline: 1.0811x; 1.0811x over previous
import functools

import jax
import jax.numpy as jnp
from jax import lax
from jax.experimental import pallas as pl
from jax.experimental.pallas import tpu as pltpu
from jax.experimental.pallas import tpu_sc as plsc

_DIM = 1024
_SCALE = _DIM ** (-0.5)
_NC, _NS = 2, 16
_NW = _NC * _NS
_LANES = 16

_CHUNK_ROWS = 8
_COL_VREGS = _DIM // _LANES
_RING = 4


@functools.partial(jax.jit, static_argnums=0)
def _sc_scaled_copy(rows, emb):
    rows_per_w = rows // _NW
    n_chunks = rows_per_w // _CHUNK_ROWS
    assert n_chunks % _RING == 0 and n_chunks >= 2 * _RING

    mesh = plsc.VectorSubcoreMesh(
        core_axis_name="c", subcore_axis_name="s",
        num_cores=_NC, num_subcores=_NS)

    @functools.partial(
        pl.kernel,
        out_type=jax.ShapeDtypeStruct((rows, _DIM), jnp.float32),
        mesh=mesh,
        scratch_types=[
            pltpu.VMEM((_CHUNK_ROWS, _DIM), jnp.float32),
            pltpu.VMEM((_CHUNK_ROWS, _DIM), jnp.float32),
            pltpu.VMEM((_CHUNK_ROWS, _DIM), jnp.float32),
            pltpu.VMEM((_CHUNK_ROWS, _DIM), jnp.float32),
            pltpu.SemaphoreType.DMA,
            pltpu.SemaphoreType.DMA,
        ],
    )
    def scale_kernel(emb_hbm, out_hbm, b0, b1, b2, b3, sem_in, sem_out):
        wid = lax.axis_index("s") * _NC + lax.axis_index("c")
        base = wid * rows_per_w
        bufs = [b0, b1, b2, b3]

        def issue_in(c, b):
            pltpu.async_copy(
                emb_hbm.at[pl.ds(base + c * _CHUNK_ROWS, _CHUNK_ROWS)],
                bufs[b], sem_in)

        def issue_out(c, b):
            pltpu.async_copy(
                bufs[b], out_hbm.at[pl.ds(base + c * _CHUNK_ROWS, _CHUNK_ROWS)],
                sem_out)

        def wait_one(sem):
            pltpu.make_async_copy(
                emb_hbm.at[pl.ds(0, _CHUNK_ROWS)], bufs[0], sem).wait()

        def compute(b):
            buf = bufs[b]

            @plsc.parallel_loop(0, _COL_VREGS, unroll=8)
            def _(i):
                sl = pl.ds(i * _LANES, _LANES)
                for r in range(_CHUNK_ROWS):
                    buf[r, sl] = buf[r, sl] * _SCALE

        issue_in(0, 0)
        issue_in(1, 1)
        for c in range(_RING):
            if c >= 2:
                wait_one(sem_out)
            issue_in(c + 2, (c + 2) % _RING)
            wait_one(sem_in)
            compute(c % _RING)
            issue_out(c, c % _RING)

        def ring_body(g, carry):
            for b in range(_RING):
                c = _RING * g + b
                wait_one(sem_out)
                issue_in(c + 2, (b + 2) % _RING)
                wait_one(sem_in)
                compute(b)
                issue_out(c, b)
            return carry

        lax.fori_loop(1, n_chunks // _RING - 1, ring_body, 0)

        for c in range(n_chunks - _RING, n_chunks):
            b = c % _RING
            if c + 2 < n_chunks:
                wait_one(sem_out)
                issue_in(c + 2, (b + 2) % _RING)
            wait_one(sem_in)
            compute(b)
            issue_out(c, b)
        for _ in range(_RING):
            wait_one(sem_out)

    return scale_kernel(emb)


def kernel(x, emb):
    seq_len = x.shape[1]
    return _sc_scaled_copy(seq_len, emb[:seq_len])


# speedup vs baseline: 1.1309x; 1.0460x over previous
import functools

import jax
import jax.numpy as jnp
from jax import lax
from jax.experimental import pallas as pl
from jax.experimental.pallas import tpu as pltpu
from jax.experimental.pallas import tpu_sc as plsc

_DIM = 1024
_SCALE = _DIM ** (-0.5)
_NC, _NS = 2, 16
_NW = _NC * _NS
_LANES = 16

_CHUNK_ROWS = 8
_COL_VREGS = _DIM // _LANES
_RING = 4


@functools.partial(jax.jit, static_argnums=0)
def _sc_scaled_copy(rows, emb):
    rows_per_w = rows // _NW
    n_chunks = rows_per_w // _CHUNK_ROWS
    assert n_chunks % _RING == 0 and n_chunks >= 2 * _RING

    mesh = plsc.VectorSubcoreMesh(
        core_axis_name="c", subcore_axis_name="s",
        num_cores=_NC, num_subcores=_NS)

    @functools.partial(
        pl.kernel,
        out_type=jax.ShapeDtypeStruct((rows, _DIM), jnp.float32),
        mesh=mesh,
        scratch_types=[
            pltpu.VMEM((_CHUNK_ROWS, _DIM), jnp.float32),
            pltpu.VMEM((_CHUNK_ROWS, _DIM), jnp.float32),
            pltpu.VMEM((_CHUNK_ROWS, _DIM), jnp.float32),
            pltpu.VMEM((_CHUNK_ROWS, _DIM), jnp.float32),
            pltpu.SemaphoreType.DMA,
            pltpu.SemaphoreType.DMA,
        ],
    )
    def scale_kernel(emb_hbm, out_hbm, b0, b1, b2, b3, sem_in, sem_out):
        wid = lax.axis_index("s") * _NC + lax.axis_index("c")
        base = wid * rows_per_w
        bufs = [b0, b1, b2, b3]

        def issue_in(c, b):
            pltpu.async_copy(
                emb_hbm.at[pl.ds(base + c * _CHUNK_ROWS, _CHUNK_ROWS)],
                bufs[b], sem_in)

        def issue_out(c, b):
            pltpu.async_copy(
                bufs[b], out_hbm.at[pl.ds(base + c * _CHUNK_ROWS, _CHUNK_ROWS)],
                sem_out)

        def wait_one(sem):
            pltpu.make_async_copy(
                emb_hbm.at[pl.ds(0, _CHUNK_ROWS)], bufs[0], sem).wait()

        def compute(b):
            buf = bufs[b]

            @plsc.parallel_loop(0, _COL_VREGS, unroll=8)
            def _(i):
                sl = pl.ds(i * _LANES, _LANES)
                for r in range(_CHUNK_ROWS):
                    buf[r, sl] = buf[r, sl] * _SCALE

        issue_in(0, 0)
        issue_in(1, 1)

        def ring_body(g, carry):
            for b in range(_RING):
                c = _RING * g + b

                @pl.when(jnp.logical_and(c >= 2, c + 2 < n_chunks))
                def _():
                    wait_one(sem_out)

                @pl.when(c + 2 < n_chunks)
                def _():
                    issue_in(c + 2, (b + 2) % _RING)

                wait_one(sem_in)
                compute(b)
                issue_out(c, b)
            return carry

        lax.fori_loop(0, n_chunks // _RING, ring_body, 0)

        for _ in range(_RING):
            wait_one(sem_out)

    return scale_kernel(emb)


def kernel(x, emb):
    seq_len = x.shape[1]
    return _sc_scaled_copy(seq_len, emb[:seq_len])


# speedup vs baseline: 1.1507x; 1.0175x over previous
import functools

import jax
import jax.numpy as jnp
from jax import lax
from jax.experimental import pallas as pl
from jax.experimental.pallas import tpu as pltpu
from jax.experimental.pallas import tpu_sc as plsc

_DIM = 1024
_SCALE = _DIM ** (-0.5)
_NC, _NS = 2, 16
_NW = _NC * _NS
_LANES = 16

_CHUNK_ROWS = 8
_COL_VREGS = _DIM // _LANES
_RING = 4


@functools.partial(jax.jit, static_argnums=0)
def _sc_scaled_copy(rows, emb):
    rows_per_w = rows // _NW
    n_chunks = rows_per_w // _CHUNK_ROWS
    assert n_chunks % _RING == 0 and n_chunks >= 2 * _RING

    mesh = plsc.VectorSubcoreMesh(
        core_axis_name="c", subcore_axis_name="s",
        num_cores=_NC, num_subcores=_NS)

    @functools.partial(
        pl.kernel,
        out_type=jax.ShapeDtypeStruct((rows, _DIM), jnp.float32),
        mesh=mesh,
        scratch_types=[
            pltpu.VMEM((_RING, _CHUNK_ROWS, _DIM), jnp.float32),
            pltpu.SemaphoreType.DMA,
            pltpu.SemaphoreType.DMA,
        ],
    )
    def scale_kernel(emb_hbm, out_hbm, bufs, sem_in, sem_out):
        wid = lax.axis_index("s") * _NC + lax.axis_index("c")
        base = wid * rows_per_w

        def issue_in(c, b):
            pltpu.async_copy(
                emb_hbm.at[pl.ds(base + c * _CHUNK_ROWS, _CHUNK_ROWS)],
                bufs.at[b], sem_in)

        def issue_out(c, b):
            pltpu.async_copy(
                bufs.at[b], out_hbm.at[pl.ds(base + c * _CHUNK_ROWS, _CHUNK_ROWS)],
                sem_out)

        def wait_one(sem):
            pltpu.make_async_copy(
                emb_hbm.at[pl.ds(0, _CHUNK_ROWS)], bufs.at[0], sem).wait()

        def compute(b):
            @plsc.parallel_loop(0, _COL_VREGS, unroll=8)
            def _(i):
                sl = pl.ds(i * _LANES, _LANES)
                for r in range(_CHUNK_ROWS):
                    bufs[b, r, sl] = bufs[b, r, sl] * _SCALE

        issue_in(0, 0)
        issue_in(1, 1)

        def ring_body(c, carry):
            b = lax.rem(c, _RING)

            @pl.when(jnp.logical_and(c >= 2, c + 2 < n_chunks))
            def _():
                wait_one(sem_out)

            @pl.when(c + 2 < n_chunks)
            def _():
                issue_in(c + 2, lax.rem(c + 2, _RING))

            wait_one(sem_in)
            compute(b)
            issue_out(c, b)
            return carry

        lax.fori_loop(0, n_chunks, ring_body, 0)

        for _ in range(_RING):
            wait_one(sem_out)

    return scale_kernel(emb)


def kernel(x, emb):
    seq_len = x.shape[1]
    return _sc_scaled_copy(seq_len, emb[:seq_len])
